# baseline (speedup 1.0000x reference)
import functools

import jax
import jax.numpy as jnp
from jax import lax
from jax.experimental import pallas as pl
from jax.experimental.pallas import tpu as pltpu
from jax.experimental.pallas import tpu_sc as plsc

_EMBED_DIM = 32
_CHUNK = 128
_K = 16
_GROUP = _CHUNK * _K


@functools.lru_cache(maxsize=None)
def _build_gather(B):
    info = plsc.get_sparse_core_info()
    nc, ns = info.num_cores, info.num_subcores
    nw = nc * ns
    assert B % (nw * _GROUP) == 0
    b_per_w = B // nw
    n_groups = b_per_w // _GROUP
    rows_per_w = b_per_w // _CHUNK

    mesh = plsc.VectorSubcoreMesh(core_axis_name="c", subcore_axis_name="s")

    @functools.partial(
        pl.kernel,
        mesh=mesh,
        out_type=jax.ShapeDtypeStruct((B, _EMBED_DIM), jnp.float32),
        compiler_params=pltpu.CompilerParams(use_tc_tiling_on_sc=False),
        scratch_types=[
            pltpu.VMEM((_K, _CHUNK), jnp.int32),
            pltpu.VMEM((_GROUP, _EMBED_DIM), jnp.float32),
            pltpu.SemaphoreType.DMA,
        ],
    )
    def embed(idx_hbm, table_hbm, out_hbm, idx_v, rows_v, gsem):
        wid = lax.axis_index("s") * nc + lax.axis_index("c")
        row_base = wid * rows_per_w
        out_base = wid * b_per_w

        def body(g, carry):
            pltpu.sync_copy(idx_hbm.at[pl.ds(row_base + g * _K, _K)], idx_v)
            cps = [
                pltpu.async_copy(
                    table_hbm.at[idx_v.at[j]],
                    rows_v.at[pl.ds(j * _CHUNK, _CHUNK)],
                    gsem,
                )
                for j in range(_K)
            ]
            for cp in cps:
                cp.wait()
            pltpu.sync_copy(
                rows_v, out_hbm.at[pl.ds(out_base + g * _GROUP, _GROUP)]
            )
            return carry

        lax.fori_loop(0, n_groups, body, 0)

    return embed


@functools.lru_cache(maxsize=None)
def _build_relayout(batch, hist):
    rows_per_step = 16
    grid = batch // rows_per_step

    def body(in_ref, out_ref):
        out_ref[...] = in_ref[...].reshape(rows_per_step, hist, _EMBED_DIM)

    return pl.pallas_call(
        body,
        grid=(grid,),
        in_specs=[
            pl.BlockSpec((rows_per_step * hist, _EMBED_DIM), lambda i: (i, 0))
        ],
        out_specs=pl.BlockSpec(
            (rows_per_step, hist, _EMBED_DIM), lambda i: (i, 0, 0)
        ),
        out_shape=jax.ShapeDtypeStruct((batch, hist, _EMBED_DIM), jnp.float32),
    )


def kernel(x, w):
    batch, hist = x.shape
    B = batch * hist
    idx = x.reshape(B // _CHUNK, _CHUNK).astype(jnp.int32)
    flat = _build_gather(B)(idx, w)
    return _build_relayout(batch, hist)(flat)


# speedup vs baseline: 2.9095x; 2.9095x over previous
import functools

import jax
import jax.numpy as jnp
from jax import lax
from jax.experimental import pallas as pl
from jax.experimental.pallas import tpu as pltpu
from jax.experimental.pallas import tpu_sc as plsc

_EMBED_DIM = 32
_PAD_DIM = 128
_ROWS_PER_UNIT = 4


@functools.lru_cache(maxsize=None)
def _build_gather(batch, hist):
    info = plsc.get_sparse_core_info()
    nc, ns = info.num_cores, info.num_subcores
    nw = nc * ns
    ipu = _ROWS_PER_UNIT * hist
    assert batch % (nw * _ROWS_PER_UNIT) == 0
    rows_per_w = batch // nw
    n_units = rows_per_w // _ROWS_PER_UNIT
    chunks = []
    off = 0
    while off < ipu:
        sz = min(128, ipu - off)
        chunks.append((off, sz))
        off += sz

    mesh = plsc.VectorSubcoreMesh(core_axis_name="c", subcore_axis_name="s")

    @functools.partial(
        pl.kernel,
        mesh=mesh,
        out_type=jax.ShapeDtypeStruct((batch, hist, _PAD_DIM), jnp.float32),
        compiler_params=pltpu.CompilerParams(use_tc_tiling_on_sc=False),
        scratch_types=[
            pltpu.VMEM((2, ipu), jnp.int32),
            pltpu.VMEM((2, ipu, _EMBED_DIM), jnp.float32),
            pltpu.SemaphoreType.DMA,
            pltpu.SemaphoreType.DMA,
            pltpu.SemaphoreType.DMA,
            pltpu.SemaphoreType.DMA,
        ],
    )
    def embed(idx_hbm, table_hbm, out_hbm, idx_v, rows_v, g0, g1, w0, w1):
        wid = lax.axis_index("s") * nc + lax.axis_index("c")
        idx_base = wid * rows_per_w * hist
        b_base = wid * rows_per_w
        gsems = (g0, g1)
        wsems = (w0, w1)

        def unit_bytes_wait(sem):
            pltpu.make_async_copy(
                table_hbm.at[pl.ds(0, ipu)], rows_v.at[0], sem
            ).wait()

        def fire_unit(u, buf):
            pltpu.sync_copy(
                idx_hbm.at[pl.ds(idx_base + u * ipu, ipu)], idx_v.at[buf]
            )
            for off, sz in chunks:
                pltpu.async_copy(
                    table_hbm.at[idx_v.at[buf].at[pl.ds(off, sz)]],
                    rows_v.at[buf].at[pl.ds(off, sz)],
                    gsems[buf],
                )

        def write_unit(u, buf):
            for k in range(_ROWS_PER_UNIT):
                b = b_base + u * _ROWS_PER_UNIT + k
                pltpu.async_copy(
                    rows_v.at[buf].at[pl.ds(k * hist, hist)],
                    out_hbm.at[b].at[:, pl.ds(0, _EMBED_DIM)],
                    wsems[buf],
                )

        def body(u, carry):
            buf = lax.rem(u, 2)

            @pl.when(jnp.logical_and(u < n_units, u >= 2))
            def _():
                lax.switch(buf, [lambda: unit_bytes_wait(w0),
                                 lambda: unit_bytes_wait(w1)])

            @pl.when(u < n_units)
            def _():
                lax.switch(
                    buf,
                    [lambda: fire_unit(u, 0), lambda: fire_unit(u, 1)],
                )

            @pl.when(u >= 1)
            def _():
                pbuf = lax.rem(u - 1, 2)
                lax.switch(pbuf, [lambda: unit_bytes_wait(g0),
                                  lambda: unit_bytes_wait(g1)])
                lax.switch(
                    pbuf,
                    [lambda: write_unit(u - 1, 0),
                     lambda: write_unit(u - 1, 1)],
                )

            return carry

        lax.fori_loop(0, n_units + 1, body, 0)
        unit_bytes_wait(w0)
        unit_bytes_wait(w1)

    return embed


def kernel(x, w):
    batch, hist = x.shape
    idx = x.reshape(-1).astype(jnp.int32)
    out3 = _build_gather(batch, hist)(idx, w)
    return out3[:, :, :_EMBED_DIM]


# speedup vs baseline: 3.0430x; 1.0459x over previous
import functools

import jax
import jax.numpy as jnp
from jax import lax
from jax.experimental import pallas as pl
from jax.experimental.pallas import tpu as pltpu
from jax.experimental.pallas import tpu_sc as plsc

_EMBED_DIM = 32
_PAD_DIM = 128
_ROWS_PER_UNIT = 8


@functools.lru_cache(maxsize=None)
def _build_gather(batch, hist):
    info = plsc.get_sparse_core_info()
    nc, ns = info.num_cores, info.num_subcores
    nw = nc * ns
    ipu = _ROWS_PER_UNIT * hist
    assert batch % (nw * _ROWS_PER_UNIT) == 0
    rows_per_w = batch // nw
    n_units = rows_per_w // _ROWS_PER_UNIT
    chunks = []
    off = 0
    while off < ipu:
        sz = min(128, ipu - off)
        chunks.append((off, sz))
        off += sz

    mesh = plsc.VectorSubcoreMesh(core_axis_name="c", subcore_axis_name="s")

    @functools.partial(
        pl.kernel,
        mesh=mesh,
        out_type=jax.ShapeDtypeStruct((batch, hist, _PAD_DIM), jnp.float32),
        compiler_params=pltpu.CompilerParams(use_tc_tiling_on_sc=False),
        scratch_types=[
            pltpu.VMEM((2, ipu), jnp.int32),
            pltpu.VMEM((2, ipu, _EMBED_DIM), jnp.float32),
            pltpu.SemaphoreType.DMA,
            pltpu.SemaphoreType.DMA,
            pltpu.SemaphoreType.DMA,
            pltpu.SemaphoreType.DMA,
            pltpu.SemaphoreType.DMA,
            pltpu.SemaphoreType.DMA,
        ],
    )
    def embed(idx_hbm, table_hbm, out_hbm, idx_v, rows_v,
              g0, g1, w0, w1, i0, i1):
        wid = lax.axis_index("s") * nc + lax.axis_index("c")
        idx_base = wid * rows_per_w * hist
        b_base = wid * rows_per_w

        def rows_bytes_wait(sem):
            pltpu.make_async_copy(
                table_hbm.at[pl.ds(0, ipu)], rows_v.at[0], sem
            ).wait()

        def idx_bytes_wait(sem):
            pltpu.make_async_copy(
                idx_hbm.at[pl.ds(0, ipu)], idx_v.at[0], sem
            ).wait()

        def fire_idx(u, buf, sem):
            pltpu.async_copy(
                idx_hbm.at[pl.ds(idx_base + u * ipu, ipu)],
                idx_v.at[buf],
                sem,
            )

        def fire_gathers(buf, sem):
            for off, sz in chunks:
                pltpu.async_copy(
                    table_hbm.at[idx_v.at[buf].at[pl.ds(off, sz)]],
                    rows_v.at[buf].at[pl.ds(off, sz)],
                    sem,
                )

        def fire_writes(u, buf, sem):
            for k in range(_ROWS_PER_UNIT):
                b = b_base + u * _ROWS_PER_UNIT + k
                pltpu.async_copy(
                    rows_v.at[buf].at[pl.ds(k * hist, hist)],
                    out_hbm.at[b].at[:, pl.ds(0, _EMBED_DIM)],
                    sem,
                )

        fire_idx(0, 0, i0)
        fire_idx(1, 1, i1)

        def body(u, carry):
            buf = lax.rem(u, 2)

            @pl.when(jnp.logical_and(u < n_units, u >= 2))
            def _():
                lax.switch(buf, [lambda: rows_bytes_wait(w0),
                                 lambda: rows_bytes_wait(w1)])

            @pl.when(u < n_units)
            def _():
                lax.switch(buf, [lambda: idx_bytes_wait(i0),
                                 lambda: idx_bytes_wait(i1)])
                lax.switch(buf, [lambda: fire_gathers(0, g0),
                                 lambda: fire_gathers(1, g1)])

            @pl.when(u >= 1)
            def _():
                pbuf = lax.rem(u - 1, 2)
                lax.switch(pbuf, [lambda: rows_bytes_wait(g0),
                                  lambda: rows_bytes_wait(g1)])
                @pl.when(u + 1 < n_units)
                def _():
                    lax.switch(pbuf, [lambda: fire_idx(u + 1, 0, i0),
                                      lambda: fire_idx(u + 1, 1, i1)])

                lax.switch(pbuf, [lambda: fire_writes(u - 1, 0, w0),
                                  lambda: fire_writes(u - 1, 1, w1)])

            return carry

        lax.fori_loop(0, n_units + 1, body, 0)
        rows_bytes_wait(w0)
        rows_bytes_wait(w1)

    return embed


def kernel(x, w):
    batch, hist = x.shape
    idx = x.reshape(-1).astype(jnp.int32)
    out3 = _build_gather(batch, hist)(idx, w)
    return out3[:, :, :_EMBED_DIM]
